# baseline (speedup 1.0000x reference)
import jax
import jax.numpy as jnp
from jax import lax
from jax.experimental import pallas as pl
from jax.experimental.pallas import tpu as pltpu
from jax.experimental.pallas import tpu_sc as plsc

NC = 2
NS = 16
NW = NC * NS
LANES = 16
EMBED = 64
NBUF = 4
SCALE = 8.0
SPLIT = 120


def _make_sc_kernel(b, t):
    mesh = plsc.VectorSubcoreMesh(core_axis_name="c", subcore_axis_name="s")
    rows_per_w = b // NW

    def body(idx_hbm, table_hbm, out_hbm, idx_v, in_v, out_v, *sems):
        gsems = sems[:NBUF]
        ssems = sems[NBUF:]
        wid = lax.axis_index("s") * NC + lax.axis_index("c")
        row0 = wid * rows_per_w
        pltpu.sync_copy(idx_hbm.at[pl.ds(row0, rows_per_w)], idx_v)

        def gather(r, slot):
            pltpu.async_copy(table_hbm.at[idx_v.at[r, pl.ds(0, SPLIT)]],
                             in_v.at[slot, pl.ds(0, SPLIT)], gsems[slot])
            pltpu.async_copy(table_hbm.at[idx_v.at[r, pl.ds(SPLIT, t - SPLIT)]],
                             in_v.at[slot, pl.ds(SPLIT, t - SPLIT)], gsems[slot])

        def gather_wait(slot):
            pltpu.make_async_copy(
                table_hbm.at[idx_v.at[0, pl.ds(0, SPLIT)]],
                in_v.at[slot, pl.ds(0, SPLIT)], gsems[slot]).wait()
            pltpu.make_async_copy(
                table_hbm.at[idx_v.at[0, pl.ds(SPLIT, t - SPLIT)]],
                in_v.at[slot, pl.ds(SPLIT, t - SPLIT)], gsems[slot]).wait()

        def scatter(r, slot):
            pltpu.async_copy(out_v.at[slot], out_hbm.at[row0 + r], ssems[slot])

        def scatter_wait(r, slot):
            pltpu.make_async_copy(
                out_v.at[slot], out_hbm.at[row0 + r], ssems[slot]).wait()

        for slot in range(NBUF):
            gather(slot, slot)

        n_groups = rows_per_w // NBUF

        def group_body(g, carry):
            for slot in range(NBUF):
                r = g * NBUF + slot
                gather_wait(slot)

                @pl.when(g >= 1)
                def _():
                    scatter_wait(r - NBUF, slot)

                @plsc.parallel_loop(0, t, 1, unroll=4)
                def _(i):
                    for k in range(EMBED // LANES):
                        sl = pl.ds(k * LANES, LANES)
                        out_v[slot, i, sl] = in_v[slot, i, sl] * SCALE

                @pl.when(g < n_groups - 1)
                def _():
                    gather(r + NBUF, slot)

                scatter(r, slot)
            return carry

        lax.fori_loop(0, n_groups, group_body, 0)

        for slot in range(NBUF):
            scatter_wait(rows_per_w - NBUF + slot, slot)

    return pl.kernel(
        body,
        out_type=jax.ShapeDtypeStruct((b, t, EMBED), jnp.float32),
        mesh=mesh,
        scratch_types=[
            pltpu.VMEM((rows_per_w, t), jnp.int32),
            pltpu.VMEM((NBUF, t, EMBED), jnp.float32),
            pltpu.VMEM((NBUF, t, EMBED), jnp.float32),
        ] + [pltpu.SemaphoreType.DMA] * (2 * NBUF),
        compiler_params=pltpu.CompilerParams(use_tc_tiling_on_sc=False),
    )


def kernel(input, table):
    b, t = input.shape
    return _make_sc_kernel(b, t)(input.astype(jnp.int32), table)
